# baseline (speedup 1.0000x reference)
import numpy as np
import jax
import jax.numpy as jnp
from jax.experimental import pallas as pl
from jax.experimental.pallas import tpu as pltpu

_NODE_FEATS = [(119, False), (4, False), (11, False), (12, False), (9, False),
               (5, False), (8, False), (2, True), (2, True)]
_EDGE_FEATS = [(22, False), (6, False), (2, True)]


def _build_consts(feats):
    W = sum(1 if ib else d for d, ib in feats)
    nf = len(feats)
    sel = np.zeros((nf, W), np.int32)
    dmax = np.zeros((1, W), np.int32)
    tgt = np.zeros((1, W), np.int32)
    isb = np.zeros((1, W), np.int32)
    c = 0
    for i, (d, ib) in enumerate(feats):
        if ib:
            sel[i, c] = 1
            isb[0, c] = 1
            c += 1
        else:
            for t in range(d):
                sel[i, c] = 1
                tgt[0, c] = t
                dmax[0, c] = d - 1
                c += 1
    assert c == W
    return sel, dmax, tgt, isb, W


def _oh_kernel(x_ref, sel_ref, dmax_ref, tgt_ref, isb_ref, o_ref):
    nf = sel_ref.shape[0]
    acc = x_ref[:, 0:1] * sel_ref[0:1, :]
    for i in range(1, nf):
        acc = acc + x_ref[:, i:i + 1] * sel_ref[i:i + 1, :]
    clipped = jnp.minimum(jnp.maximum(acc, 0), dmax_ref[0:1, :])
    oh = ((clipped == tgt_ref[0:1, :]) & (acc != -1)).astype(jnp.float32)
    bv = jnp.where(acc == -1, 0, acc).astype(jnp.float32)
    o_ref[...] = jnp.where(isb_ref[0:1, :] != 0, bv, oh)


def _encode(t, feats, block_rows):
    sel, dmax, tgt, isb, W = _build_consts(feats)
    N, nf = t.shape
    assert N % block_rows == 0
    grid = (N // block_rows,)
    full = lambda i: (0, 0)
    return pl.pallas_call(
        _oh_kernel,
        grid=grid,
        in_specs=[
            pl.BlockSpec((block_rows, nf), lambda i: (i, 0)),
            pl.BlockSpec((nf, W), full),
            pl.BlockSpec((1, W), full),
            pl.BlockSpec((1, W), full),
            pl.BlockSpec((1, W), full),
        ],
        out_specs=pl.BlockSpec((block_rows, W), lambda i: (i, 0)),
        out_shape=jax.ShapeDtypeStruct((N, W), jnp.float32),
        compiler_params=pltpu.CompilerParams(
            dimension_semantics=("parallel",)),
    )(t, jnp.asarray(sel), jnp.asarray(dmax), jnp.asarray(tgt),
      jnp.asarray(isb))


@jax.jit
def kernel(x, e):
    x_onehot = _encode(x, _NODE_FEATS, 2000)
    e_onehot = _encode(e, _EDGE_FEATS, 8000)
    return (x_onehot, e_onehot)
